# Initial kernel scaffold; baseline (speedup 1.0000x reference)
import jax
import jax.numpy as jnp
from jax.experimental import pallas as pl


def kernel(original_points, data, perm, params):
    raise NotImplementedError("write your pallas kernel here")


# speedup vs baseline: 1.9374x; 1.9374x over previous
import functools

import jax
import jax.numpy as jnp
from jax.experimental import pallas as pl

F32 = jnp.float32
_INTERPRET = False

_BIG = 1 << 30


def _dot(a, b):
    return jnp.dot(a, b, preferred_element_type=F32)


def _dotx(a, b):
    return jnp.dot(a, b, preferred_element_type=F32,
                   precision=jax.lax.Precision.HIGHEST)


def _relu(x):
    return jnp.maximum(x, 0.0)


def _softmax_axis(x, axis):
    m = jnp.max(x, axis=axis, keepdims=True)
    e = jnp.exp(x - m)
    return e / jnp.sum(e, axis=axis, keepdims=True)


def _ptl3_body(x, pos, P):
    T = x.shape[0]
    qkv = _dot(x.reshape(T * 16, 3), P[0]).reshape(T, 16, 9)
    q, k, v = qkv[:, :, 0:3], qkv[:, :, 3:6], qkv[:, :, 6:9]
    rel = pos[:, :, None, :] - pos[:, None, :, :]
    h = _relu(_dot(rel.reshape(T * 256, 3), P[1]) + P[2])
    emb = (_dot(h, P[3]) + P[4]).reshape(T, 16, 16, 3)
    qk = q[:, :, None, :] - k[:, None, :, :]
    vb = v[:, None, :, :] + emb
    h2 = _relu(_dot((qk + emb).reshape(T * 256, 3), P[5]) + P[6])
    sim = (_dot(h2, P[7]) + P[8]).reshape(T, 16, 16, 3)
    attn = _softmax_axis(sim, 2)
    return jnp.sum(attn * vb, axis=2)


def _topk16_gather(dist, feats, clamp=None):
    R, N = dist.shape
    C = feats.shape[0]
    neg = -dist
    jcol = jax.lax.broadcasted_iota(jnp.int32, (R, N), 1)
    jg = jcol if clamp is None else jax.lax.broadcasted_iota(jnp.int32, (R, C), 1)
    outs = []
    for _ in range(16):
        m = jnp.max(neg, axis=1, keepdims=True)
        cand = jnp.where(neg == m, jcol, _BIG)
        jmin = jnp.min(cand, axis=1, keepdims=True)
        neg = jnp.where(jcol == jmin, -jnp.inf, neg)
        if clamp is not None:
            jmin = jnp.minimum(jmin, clamp)
        oh = (jg == jmin).astype(F32)
        outs.append(_dotx(oh, feats))
    return jnp.stack(outs, axis=1)


def _pair_dist(p, pT):
    dx = p[:, 0:1] - pT[0:1, :]
    dy = p[:, 1:2] - pT[1:2, :]
    dz = p[:, 2:3] - pT[2:3, :]
    return jnp.sqrt(dx * dx + dy * dy + dz * dz)


def _full(shape):
    return pl.BlockSpec(shape, lambda *_: (0,) * len(shape))


def _attn_param_list(p):
    return [p['to_qkv'],
            p['pos_w1'], p['pos_b1'].reshape(1, -1),
            p['pos_w2'], p['pos_b2'].reshape(1, -1),
            p['attn_w1'], p['attn_b1'].reshape(1, -1),
            p['attn_w2'], p['attn_b2'].reshape(1, -1)]


def _small_attn_body(nP, nM, *refs):
    x_ref, pos_ref = refs[0], refs[1]
    i = 2 + (1 if nM == 6 else 0)
    P = [r[...] for r in refs[i:i + nP]]
    M = [r[...] for r in refs[i + nP:i + nP + nM]]
    out_ref = refs[-1]
    a = _ptl3_body(x_ref[...], pos_ref[...], P)
    T = a.shape[0]
    o = a.reshape(T, 48)
    if nM == 4:
        o = _relu(_dot(o, M[0]) + M[1])
        o = _relu(_dot(o, M[2]) + M[3])
    elif nM == 6:
        o = jnp.concatenate([o, refs[2][...]], axis=1)
        o = _relu(_dot(o, M[0]) + M[1])
        o = _relu(_dot(o, M[2]) + M[3])
        o = _dot(o, M[4]) + M[5]
    out_ref[...] = o


def _small_attn_call(x, pos, pA, mlp=None, mlp_out=None, sa2rows=None, T=32):
    G = x.shape[0]
    params = _attn_param_list(pA)
    extra = []
    nM = 0
    if mlp is not None:
        extra = [mlp['w1'], mlp['b1'].reshape(1, -1),
                 mlp['w2'], mlp['b2'].reshape(1, -1)]
        nM = 4
    if mlp_out is not None:
        extra = [mlp_out['w1'], mlp_out['b1'].reshape(1, -1),
                 mlp_out['w2'], mlp_out['b2'].reshape(1, -1),
                 mlp_out['w3'], mlp_out['b3'].reshape(1, -1)]
        nM = 6
    in_specs = [pl.BlockSpec((T, 16, 3), lambda i: (i, 0, 0)),
                pl.BlockSpec((T, 16, 3), lambda i: (i, 0, 0))]
    args = [x, pos]
    if nM == 6:
        in_specs.append(pl.BlockSpec((T, 48), lambda i: (i, 0)))
        args.append(sa2rows)
    in_specs += [_full(p.shape) for p in params + extra]
    args += params + extra
    Fout = 1 if nM == 6 else 48
    return pl.pallas_call(
        functools.partial(_small_attn_body, 9, nM),
        grid=(G // T,),
        in_specs=in_specs,
        out_specs=pl.BlockSpec((T, Fout), lambda i: (i, 0)),
        out_shape=jax.ShapeDtypeStruct((G, Fout), F32),
        interpret=_INTERPRET,
    )(*args)


def _stage_b_body(sa1_ref, pts_ref, ptsT_ref, *rest):
    refs, out_ref = rest[:-1], rest[-1]
    (qkv_w, pw1, pb1, pw2, pb2, aw1, ab1, aw2, ab2,
     mw1, mb1, mw2, mb2) = (r[...] for r in refs)
    qi = pl.program_id(1)
    x = sa1_ref[0]
    pT = ptsT_ref[0]
    p = pts_ref[0]
    kv = _dot(x, qkv_w)
    k, v = kv[:, 48:96], kv[:, 96:144]
    xq = sa1_ref[0, pl.ds(qi * 128, 128), :]
    pq = pts_ref[0, pl.ds(qi * 128, 128), :]
    q = _dot(xq, qkv_w)[:, 0:48]
    dist = _pair_dist(pq, pT)
    feats = jnp.concatenate([k, v, p], axis=1)
    g = _topk16_gather(dist, feats)
    gk, gv, gp = g[:, :, 0:48], g[:, :, 48:96], g[:, :, 96:99]
    grel = pq[:, None, :] - gp
    h = _relu(_dot(grel.reshape(128 * 16, 3), pw1) + pb1)
    emb = (_dot(h, pw2) + pb2).reshape(128, 16, 48)
    qkr = q[:, None, :] - gk
    vg = gv + emb
    h2 = _relu(_dot((qkr + emb).reshape(128 * 16, 48), aw1) + ab1)
    sim = (_dot(h2, aw2) + ab2).reshape(128, 16, 48)
    attn = _softmax_axis(sim, 1)
    o = jnp.sum(attn * vg, axis=1)
    o = _relu(_dot(o, mw1) + mb1)
    o = _relu(_dot(o, mw2) + mb2)
    out_ref[0] = o


def _call_b(sa1, pts, ptsT, pA, pM):
    B = sa1.shape[0]
    params = _attn_param_list(pA) + [pM['w1'], pM['b1'].reshape(1, -1),
                                     pM['w2'], pM['b2'].reshape(1, -1)]
    in_specs = [pl.BlockSpec((1, 512, 48), lambda b, q: (b, 0, 0)),
                pl.BlockSpec((1, 512, 3), lambda b, q: (b, 0, 0)),
                pl.BlockSpec((1, 3, 512), lambda b, q: (b, 0, 0))]
    in_specs += [_full(p.shape) for p in params]
    return pl.pallas_call(
        _stage_b_body,
        grid=(B, 4),
        in_specs=in_specs,
        out_specs=pl.BlockSpec((1, 128, 48), lambda b, q: (b, q, 0)),
        out_shape=jax.ShapeDtypeStruct((B, 512, 48), F32),
        interpret=_INTERPRET,
    )(sa1, pts, ptsT, *params)


def _stage_c1_body(sa2_ref, pts_ref, ptsT_ref, perm_ref,
                   xm_ref, gp_ref, piv_ref):
    s = sa2_ref[0]
    p = pts_ref[0]
    pT = ptsT_ref[0]
    prm = perm_ref[...]
    j512 = jax.lax.broadcasted_iota(jnp.int32, (128, 512), 1)
    ohp = (j512 == prm).astype(F32)
    piv = _dotx(ohp, p)
    dist = _pair_dist(piv, pT)
    feats = jnp.concatenate([s, p], axis=1)
    g = _topk16_gather(dist, feats)
    gs, gp = g[:, :, 0:48], g[:, :, 48:51]
    xm_ref[0] = jnp.max(gs.reshape(128, 16, 3, 16), axis=-1)
    gp_ref[0] = gp
    piv_ref[0] = piv


def _call_c1(sa2, pts, ptsT, permc):
    B = sa2.shape[0]
    in_specs = [pl.BlockSpec((1, 512, 48), lambda b: (b, 0, 0)),
                pl.BlockSpec((1, 512, 3), lambda b: (b, 0, 0)),
                pl.BlockSpec((1, 3, 512), lambda b: (b, 0, 0)),
                _full((128, 1))]
    return pl.pallas_call(
        _stage_c1_body,
        grid=(B,),
        in_specs=in_specs,
        out_specs=[pl.BlockSpec((1, 128, 16, 3), lambda b: (b, 0, 0, 0)),
                   pl.BlockSpec((1, 128, 16, 3), lambda b: (b, 0, 0, 0)),
                   pl.BlockSpec((1, 128, 3), lambda b: (b, 0, 0))],
        out_shape=[jax.ShapeDtypeStruct((B, 128, 16, 3), F32),
                   jax.ShapeDtypeStruct((B, 128, 16, 3), F32),
                   jax.ShapeDtypeStruct((B, 128, 3), F32)],
        interpret=_INTERPRET,
    )(sa2, pts, ptsT, permc)


def _stage_d_body(sa3_ref, piv_ref, *rest):
    refs, out_ref = rest[:-1], rest[-1]
    (qkv_w, pw1, pb1, pw2, pb2, aw1, ab1, aw2, ab2) = (r[...] for r in refs)
    qi = pl.program_id(1)
    x = sa3_ref[0]
    piv = piv_ref[0]
    kv = _dot(x, qkv_w)
    k, v = kv[:, 48:96], kv[:, 96:144]
    xq = sa3_ref[0, pl.ds(qi * 32, 32), :]
    pq = piv_ref[0, pl.ds(qi * 32, 32), :]
    qq = _dot(xq, qkv_w)[:, 0:48]
    rel = pq[:, None, :] - piv[None, :, :]
    h = _relu(_dot(rel.reshape(32 * 128, 3), pw1) + pb1)
    emb = (_dot(h, pw2) + pb2).reshape(32, 128, 48)
    qk = qq[:, None, :] - k[None, :, :]
    vb = v[None, :, :] + emb
    h2 = _relu(_dot((qk + emb).reshape(32 * 128, 48), aw1) + ab1)
    sim = (_dot(h2, aw2) + ab2).reshape(32, 128, 48)
    attn = _softmax_axis(sim, 1)
    out_ref[0] = jnp.sum(attn * vb, axis=1)


def _call_d(sa3, pivot, pA):
    B = sa3.shape[0]
    params = _attn_param_list(pA)
    in_specs = [pl.BlockSpec((1, 128, 48), lambda b, q: (b, 0, 0)),
                pl.BlockSpec((1, 128, 3), lambda b, q: (b, 0, 0))]
    in_specs += [_full(p.shape) for p in params]
    return pl.pallas_call(
        _stage_d_body,
        grid=(B, 4),
        in_specs=in_specs,
        out_specs=pl.BlockSpec((1, 32, 48), lambda b, q: (b, q, 0)),
        out_shape=jax.ShapeDtypeStruct((B, 128, 48), F32),
        interpret=_INTERPRET,
    )(sa3, pivot, *params)


def _stage_e1_body(pts_ref, pivT_ref, piv_ref, sa4_ref, xm_ref, gp_ref):
    qi = pl.program_id(1)
    pvT = pivT_ref[0]
    piv = piv_ref[0]
    s4 = sa4_ref[0]
    pq = pts_ref[0, pl.ds(qi * 128, 128), :]
    pc = jnp.concatenate([pvT, jnp.zeros((3, 384), F32)], axis=1)
    dist = _pair_dist(pq, pc)
    feats = jnp.concatenate([s4, piv], axis=1)
    g = _topk16_gather(dist, feats, clamp=127)
    gs, gp = g[:, :, 0:48], g[:, :, 48:51]
    xm_ref[0] = jnp.max(gs.reshape(128, 16, 3, 16), axis=-1)
    gp_ref[0] = gp


def _call_e1(pts, pivT, pivot, sa4):
    B = pts.shape[0]
    in_specs = [pl.BlockSpec((1, 512, 3), lambda b, q: (b, 0, 0)),
                pl.BlockSpec((1, 3, 128), lambda b, q: (b, 0, 0)),
                pl.BlockSpec((1, 128, 3), lambda b, q: (b, 0, 0)),
                pl.BlockSpec((1, 128, 48), lambda b, q: (b, 0, 0))]
    return pl.pallas_call(
        _stage_e1_body,
        grid=(B, 4),
        in_specs=in_specs,
        out_specs=[pl.BlockSpec((1, 128, 16, 3), lambda b, q: (b, q, 0, 0)),
                   pl.BlockSpec((1, 128, 16, 3), lambda b, q: (b, q, 0, 0))],
        out_shape=[jax.ShapeDtypeStruct((B, 512, 16, 3), F32),
                   jax.ShapeDtypeStruct((B, 512, 16, 3), F32)],
        interpret=_INTERPRET,
    )(pts, pivT, pivot, sa4)


def kernel(original_points, data, perm, params):
    B, N, S, _ = data.shape
    x0 = data.reshape(B * N, S, 3)
    sa1m = _small_attn_call(x0, x0, params['attn1'], mlp=params['mlp'])
    ptsT = jnp.swapaxes(original_points, 1, 2)
    sa2 = _call_b(sa1m.reshape(B, N, 48), original_points, ptsT,
                  params['attn2'], params['mlp'])
    permc = perm[:128].reshape(128, 1)
    cxm, cgp, pivot = _call_c1(sa2, original_points, ptsT, permc)
    sa3 = _small_attn_call(cxm.reshape(B * 128, 16, 3),
                           cgp.reshape(B * 128, 16, 3),
                           params['attn3']).reshape(B, 128, 48)
    sa4 = _call_d(sa3, pivot, params['attn4'])
    pivT = jnp.swapaxes(pivot, 1, 2)
    xm, gp = _call_e1(original_points, pivT, pivot, sa4)
    out = _small_attn_call(xm.reshape(B * N, 16, 3),
                           gp.reshape(B * N, 16, 3),
                           params['attn5'], mlp_out=params['mlp_out'],
                           sa2rows=sa2.reshape(B * N, 48))
    return out.reshape(B, N, 1)


# speedup vs baseline: 1.9783x; 1.0211x over previous
import functools

import jax
import jax.numpy as jnp
from jax.experimental import pallas as pl

F32 = jnp.float32
_INTERPRET = False

_BIG = 1 << 30


def _dot(a, b):
    return jnp.dot(a, b, preferred_element_type=F32)


def _dotx(a, b):
    return jnp.dot(a, b, preferred_element_type=F32,
                   precision=jax.lax.Precision.HIGHEST)


def _doth(a, b_hi, b_lo):
    return (jnp.dot(a, b_hi, preferred_element_type=F32)
            + jnp.dot(a, b_lo, preferred_element_type=F32))


def _relu(x):
    return jnp.maximum(x, 0.0)


def _softmax_axis(x, axis):
    m = jnp.max(x, axis=axis, keepdims=True)
    e = jnp.exp(x - m)
    return e / jnp.sum(e, axis=axis, keepdims=True)


def _ptl3_body(x, pos, P):
    T = x.shape[0]
    qkv = _dot(x.reshape(T * 16, 3), P[0]).reshape(T, 16, 9)
    q, k, v = qkv[:, :, 0:3], qkv[:, :, 3:6], qkv[:, :, 6:9]
    rel = pos[:, :, None, :] - pos[:, None, :, :]
    h = _relu(_dot(rel.reshape(T * 256, 3), P[1]) + P[2])
    emb = (_dot(h, P[3]) + P[4]).reshape(T, 16, 16, 3)
    qk = q[:, :, None, :] - k[:, None, :, :]
    vb = v[:, None, :, :] + emb
    h2 = _relu(_dot((qk + emb).reshape(T * 256, 3), P[5]) + P[6])
    sim = (_dot(h2, P[7]) + P[8]).reshape(T, 16, 16, 3)
    attn = _softmax_axis(sim, 2)
    return jnp.sum(attn * vb, axis=2)


def _topk16_gather(dist, feats, clamp=None):
    R, N = dist.shape
    C = feats.shape[0]
    neg = -dist
    jcol = jax.lax.broadcasted_iota(jnp.int32, (R, N), 1)
    jg = jcol if clamp is None else jax.lax.broadcasted_iota(jnp.int32, (R, C), 1)
    f_hi = feats.astype(jnp.bfloat16).astype(F32)
    f_lo = feats - f_hi
    outs = []
    for _ in range(16):
        m = jnp.max(neg, axis=1, keepdims=True)
        cand = jnp.where(neg == m, jcol, _BIG)
        jmin = jnp.min(cand, axis=1, keepdims=True)
        neg = jnp.where(jcol == jmin, -jnp.inf, neg)
        if clamp is not None:
            jmin = jnp.minimum(jmin, clamp)
        oh = (jg == jmin).astype(F32)
        outs.append(_doth(oh, f_hi, f_lo))
    return jnp.stack(outs, axis=1)


def _pair_dist(p, pT):
    dx = p[:, 0:1] - pT[0:1, :]
    dy = p[:, 1:2] - pT[1:2, :]
    dz = p[:, 2:3] - pT[2:3, :]
    return jnp.sqrt(dx * dx + dy * dy + dz * dz)


def _full(shape):
    return pl.BlockSpec(shape, lambda *_: (0,) * len(shape))


def _attn_param_list(p):
    return [p['to_qkv'],
            p['pos_w1'], p['pos_b1'].reshape(1, -1),
            p['pos_w2'], p['pos_b2'].reshape(1, -1),
            p['attn_w1'], p['attn_b1'].reshape(1, -1),
            p['attn_w2'], p['attn_b2'].reshape(1, -1)]


def _small_attn_body(nP, nM, *refs):
    x_ref, pos_ref = refs[0], refs[1]
    i = 2 + (1 if nM == 6 else 0)
    P = [r[...] for r in refs[i:i + nP]]
    M = [r[...] for r in refs[i + nP:i + nP + nM]]
    out_ref = refs[-1]
    a = _ptl3_body(x_ref[...], pos_ref[...], P)
    T = a.shape[0]
    o = a.reshape(T, 48)
    if nM == 4:
        o = _relu(_dot(o, M[0]) + M[1])
        o = _relu(_dot(o, M[2]) + M[3])
    elif nM == 6:
        o = jnp.concatenate([o, refs[2][...]], axis=1)
        o = _relu(_dot(o, M[0]) + M[1])
        o = _relu(_dot(o, M[2]) + M[3])
        o = _dot(o, M[4]) + M[5]
    out_ref[...] = o


def _small_attn_call(x, pos, pA, mlp=None, mlp_out=None, sa2rows=None, T=32):
    G = x.shape[0]
    params = _attn_param_list(pA)
    extra = []
    nM = 0
    if mlp is not None:
        extra = [mlp['w1'], mlp['b1'].reshape(1, -1),
                 mlp['w2'], mlp['b2'].reshape(1, -1)]
        nM = 4
    if mlp_out is not None:
        extra = [mlp_out['w1'], mlp_out['b1'].reshape(1, -1),
                 mlp_out['w2'], mlp_out['b2'].reshape(1, -1),
                 mlp_out['w3'], mlp_out['b3'].reshape(1, -1)]
        nM = 6
    in_specs = [pl.BlockSpec((T, 16, 3), lambda i: (i, 0, 0)),
                pl.BlockSpec((T, 16, 3), lambda i: (i, 0, 0))]
    args = [x, pos]
    if nM == 6:
        in_specs.append(pl.BlockSpec((T, 48), lambda i: (i, 0)))
        args.append(sa2rows)
    in_specs += [_full(p.shape) for p in params + extra]
    args += params + extra
    Fout = 1 if nM == 6 else 48
    return pl.pallas_call(
        functools.partial(_small_attn_body, 9, nM),
        grid=(G // T,),
        in_specs=in_specs,
        out_specs=pl.BlockSpec((T, Fout), lambda i: (i, 0)),
        out_shape=jax.ShapeDtypeStruct((G, Fout), F32),
        interpret=_INTERPRET,
    )(*args)


def _stage_b_body(sa1_ref, pts_ref, ptsT_ref, *rest):
    refs, out_ref = rest[:-1], rest[-1]
    (qkv_w, pw1, pb1, pw2, pb2, aw1, ab1, aw2, ab2,
     mw1, mb1, mw2, mb2) = (r[...] for r in refs)
    qi = pl.program_id(1)
    x = sa1_ref[0]
    pT = ptsT_ref[0]
    p = pts_ref[0]
    kv = _dot(x, qkv_w)
    k, v = kv[:, 48:96], kv[:, 96:144]
    xq = sa1_ref[0, pl.ds(qi * 128, 128), :]
    pq = pts_ref[0, pl.ds(qi * 128, 128), :]
    q = _dot(xq, qkv_w)[:, 0:48]
    dist = _pair_dist(pq, pT)
    feats = jnp.concatenate([k, v, p], axis=1)
    g = _topk16_gather(dist, feats)
    gk, gv, gp = g[:, :, 0:48], g[:, :, 48:96], g[:, :, 96:99]
    grel = pq[:, None, :] - gp
    h = _relu(_dot(grel.reshape(128 * 16, 3), pw1) + pb1)
    emb = (_dot(h, pw2) + pb2).reshape(128, 16, 48)
    qkr = q[:, None, :] - gk
    vg = gv + emb
    h2 = _relu(_dot((qkr + emb).reshape(128 * 16, 48), aw1) + ab1)
    sim = (_dot(h2, aw2) + ab2).reshape(128, 16, 48)
    attn = _softmax_axis(sim, 1)
    o = jnp.sum(attn * vg, axis=1)
    o = _relu(_dot(o, mw1) + mb1)
    o = _relu(_dot(o, mw2) + mb2)
    out_ref[0] = o


def _call_b(sa1, pts, ptsT, pA, pM):
    B = sa1.shape[0]
    params = _attn_param_list(pA) + [pM['w1'], pM['b1'].reshape(1, -1),
                                     pM['w2'], pM['b2'].reshape(1, -1)]
    in_specs = [pl.BlockSpec((1, 512, 48), lambda b, q: (b, 0, 0)),
                pl.BlockSpec((1, 512, 3), lambda b, q: (b, 0, 0)),
                pl.BlockSpec((1, 3, 512), lambda b, q: (b, 0, 0))]
    in_specs += [_full(p.shape) for p in params]
    return pl.pallas_call(
        _stage_b_body,
        grid=(B, 4),
        in_specs=in_specs,
        out_specs=pl.BlockSpec((1, 128, 48), lambda b, q: (b, q, 0)),
        out_shape=jax.ShapeDtypeStruct((B, 512, 48), F32),
        interpret=_INTERPRET,
    )(sa1, pts, ptsT, *params)


def _stage_c1_body(sa2_ref, pts_ref, ptsT_ref, perm_ref,
                   xm_ref, gp_ref, piv_ref):
    s = sa2_ref[0]
    p = pts_ref[0]
    pT = ptsT_ref[0]
    prm = perm_ref[...]
    j512 = jax.lax.broadcasted_iota(jnp.int32, (128, 512), 1)
    ohp = (j512 == prm).astype(F32)
    piv = _dotx(ohp, p)
    dist = _pair_dist(piv, pT)
    feats = jnp.concatenate([s, p], axis=1)
    g = _topk16_gather(dist, feats)
    gs, gp = g[:, :, 0:48], g[:, :, 48:51]
    xm_ref[0] = jnp.max(gs.reshape(128, 16, 3, 16), axis=-1)
    gp_ref[0] = gp
    piv_ref[0] = piv


def _call_c1(sa2, pts, ptsT, permc):
    B = sa2.shape[0]
    in_specs = [pl.BlockSpec((1, 512, 48), lambda b: (b, 0, 0)),
                pl.BlockSpec((1, 512, 3), lambda b: (b, 0, 0)),
                pl.BlockSpec((1, 3, 512), lambda b: (b, 0, 0)),
                _full((128, 1))]
    return pl.pallas_call(
        _stage_c1_body,
        grid=(B,),
        in_specs=in_specs,
        out_specs=[pl.BlockSpec((1, 128, 16, 3), lambda b: (b, 0, 0, 0)),
                   pl.BlockSpec((1, 128, 16, 3), lambda b: (b, 0, 0, 0)),
                   pl.BlockSpec((1, 128, 3), lambda b: (b, 0, 0))],
        out_shape=[jax.ShapeDtypeStruct((B, 128, 16, 3), F32),
                   jax.ShapeDtypeStruct((B, 128, 16, 3), F32),
                   jax.ShapeDtypeStruct((B, 128, 3), F32)],
        interpret=_INTERPRET,
    )(sa2, pts, ptsT, permc)


def _stage_d_body(sa3_ref, piv_ref, *rest):
    refs, out_ref = rest[:-1], rest[-1]
    (qkv_w, pw1, pb1, pw2, pb2, aw1, ab1, aw2, ab2) = (r[...] for r in refs)
    qi = pl.program_id(1)
    x = sa3_ref[0]
    piv = piv_ref[0]
    kv = _dot(x, qkv_w)
    k, v = kv[:, 48:96], kv[:, 96:144]
    xq = sa3_ref[0, pl.ds(qi * 32, 32), :]
    pq = piv_ref[0, pl.ds(qi * 32, 32), :]
    qq = _dot(xq, qkv_w)[:, 0:48]
    rel = pq[:, None, :] - piv[None, :, :]
    h = _relu(_dot(rel.reshape(32 * 128, 3), pw1) + pb1)
    emb = (_dot(h, pw2) + pb2).reshape(32, 128, 48)
    qk = qq[:, None, :] - k[None, :, :]
    vb = v[None, :, :] + emb
    h2 = _relu(_dot((qk + emb).reshape(32 * 128, 48), aw1) + ab1)
    sim = (_dot(h2, aw2) + ab2).reshape(32, 128, 48)
    attn = _softmax_axis(sim, 1)
    out_ref[0] = jnp.sum(attn * vb, axis=1)


def _call_d(sa3, pivot, pA):
    B = sa3.shape[0]
    params = _attn_param_list(pA)
    in_specs = [pl.BlockSpec((1, 128, 48), lambda b, q: (b, 0, 0)),
                pl.BlockSpec((1, 128, 3), lambda b, q: (b, 0, 0))]
    in_specs += [_full(p.shape) for p in params]
    return pl.pallas_call(
        _stage_d_body,
        grid=(B, 4),
        in_specs=in_specs,
        out_specs=pl.BlockSpec((1, 32, 48), lambda b, q: (b, q, 0)),
        out_shape=jax.ShapeDtypeStruct((B, 128, 48), F32),
        interpret=_INTERPRET,
    )(sa3, pivot, *params)


def _stage_e1_body(pts_ref, pivT_ref, piv_ref, sa4_ref, xm_ref, gp_ref):
    qi = pl.program_id(1)
    pvT = pivT_ref[0]
    piv = piv_ref[0]
    s4 = sa4_ref[0]
    pq = pts_ref[0, pl.ds(qi * 128, 128), :]
    pc = jnp.concatenate([pvT, jnp.zeros((3, 384), F32)], axis=1)
    dist = _pair_dist(pq, pc)
    feats = jnp.concatenate([s4, piv], axis=1)
    g = _topk16_gather(dist, feats, clamp=127)
    gs, gp = g[:, :, 0:48], g[:, :, 48:51]
    xm_ref[0] = jnp.max(gs.reshape(128, 16, 3, 16), axis=-1)
    gp_ref[0] = gp


def _call_e1(pts, pivT, pivot, sa4):
    B = pts.shape[0]
    in_specs = [pl.BlockSpec((1, 512, 3), lambda b, q: (b, 0, 0)),
                pl.BlockSpec((1, 3, 128), lambda b, q: (b, 0, 0)),
                pl.BlockSpec((1, 128, 3), lambda b, q: (b, 0, 0)),
                pl.BlockSpec((1, 128, 48), lambda b, q: (b, 0, 0))]
    return pl.pallas_call(
        _stage_e1_body,
        grid=(B, 4),
        in_specs=in_specs,
        out_specs=[pl.BlockSpec((1, 128, 16, 3), lambda b, q: (b, q, 0, 0)),
                   pl.BlockSpec((1, 128, 16, 3), lambda b, q: (b, q, 0, 0))],
        out_shape=[jax.ShapeDtypeStruct((B, 512, 16, 3), F32),
                   jax.ShapeDtypeStruct((B, 512, 16, 3), F32)],
        interpret=_INTERPRET,
    )(pts, pivT, pivot, sa4)


def kernel(original_points, data, perm, params):
    B, N, S, _ = data.shape
    x0 = data.reshape(B * N, S, 3)
    sa1m = _small_attn_call(x0, x0, params['attn1'], mlp=params['mlp'])
    ptsT = jnp.swapaxes(original_points, 1, 2)
    sa2 = _call_b(sa1m.reshape(B, N, 48), original_points, ptsT,
                  params['attn2'], params['mlp'])
    permc = perm[:128].reshape(128, 1)
    cxm, cgp, pivot = _call_c1(sa2, original_points, ptsT, permc)
    sa3 = _small_attn_call(cxm.reshape(B * 128, 16, 3),
                           cgp.reshape(B * 128, 16, 3),
                           params['attn3']).reshape(B, 128, 48)
    sa4 = _call_d(sa3, pivot, params['attn4'])
    pivT = jnp.swapaxes(pivot, 1, 2)
    xm, gp = _call_e1(original_points, pivT, pivot, sa4)
    out = _small_attn_call(xm.reshape(B * N, 16, 3),
                           gp.reshape(B * N, 16, 3),
                           params['attn5'], mlp_out=params['mlp_out'],
                           sa2rows=sa2.reshape(B * N, 48))
    return out.reshape(B, N, 1)


# speedup vs baseline: 2.1470x; 1.0853x over previous
import functools

import jax
import jax.numpy as jnp
from jax.experimental import pallas as pl

F32 = jnp.float32
_INTERPRET = False

_BIG = 1 << 30


def _dot(a, b):
    return jnp.dot(a, b, preferred_element_type=F32)


def _dotx(a, b):
    return jnp.dot(a, b, preferred_element_type=F32,
                   precision=jax.lax.Precision.HIGHEST)


def _doth(a, b_hi, b_lo):
    return (jnp.dot(a, b_hi, preferred_element_type=F32)
            + jnp.dot(a, b_lo, preferred_element_type=F32))


def _relu(x):
    return jnp.maximum(x, 0.0)


def _softmax_axis(x, axis):
    m = jnp.max(x, axis=axis, keepdims=True)
    e = jnp.exp(x - m)
    return e / jnp.sum(e, axis=axis, keepdims=True)


def _ptl3_body(x, pos, P):
    T = x.shape[0]
    qkv = _dot(x.reshape(T * 16, 3), P[0]).reshape(T, 16, 9)
    q, k, v = qkv[:, :, 0:3], qkv[:, :, 3:6], qkv[:, :, 6:9]
    posJ = jnp.swapaxes(pos, 1, 2).reshape(T, 1, 48)
    kJ = jnp.swapaxes(k, 1, 2).reshape(T, 1, 48)
    vJ = jnp.swapaxes(v, 1, 2).reshape(T, 1, 48)
    posI = jnp.broadcast_to(pos[:, :, :, None], (T, 16, 3, 16)).reshape(T, 16, 48)
    qI = jnp.broadcast_to(q[:, :, :, None], (T, 16, 3, 16)).reshape(T, 16, 48)
    rel = (posI - posJ).reshape(T * 16, 48)
    h = _relu(_dot(rel, P[1]) + P[2])
    emb = _dot(h, P[3]) + P[4]
    qk = (qI - kJ).reshape(T * 16, 48)
    vb = vJ.reshape(T, 1, 48) + emb.reshape(T, 16, 48)
    h2 = _relu(_dot(qk + emb, P[5]) + P[6])
    sim = (_dot(h2, P[7]) + P[8]).reshape(T, 16, 48)
    m = jnp.max(sim, axis=2, keepdims=True)
    e = jnp.exp(sim - m)
    esum = _dotx(e.reshape(T * 16, 48), P[9]).reshape(T, 16, 48)
    attn = e / esum
    return _dotx((attn * vb).reshape(T * 16, 48), P[10]).reshape(T, 16, 3)


def _topk16_gather(dist, feats, clamp=None):
    R, N = dist.shape
    C = feats.shape[0]
    neg = -dist
    jcol = jax.lax.broadcasted_iota(jnp.int32, (R, N), 1)
    jg = jcol if clamp is None else jax.lax.broadcasted_iota(jnp.int32, (R, C), 1)
    f_hi = feats.astype(jnp.bfloat16).astype(F32)
    f_lo = feats - f_hi
    outs = []
    for _ in range(16):
        m = jnp.max(neg, axis=1, keepdims=True)
        cand = jnp.where(neg == m, jcol, _BIG)
        jmin = jnp.min(cand, axis=1, keepdims=True)
        neg = jnp.where(jcol == jmin, -jnp.inf, neg)
        if clamp is not None:
            jmin = jnp.minimum(jmin, clamp)
        oh = (jg == jmin).astype(F32)
        outs.append(_doth(oh, f_hi, f_lo))
    return jnp.stack(outs, axis=1)


def _pair_dist(p, pT):
    dx = p[:, 0:1] - pT[0:1, :]
    dy = p[:, 1:2] - pT[1:2, :]
    dz = p[:, 2:3] - pT[2:3, :]
    return jnp.sqrt(dx * dx + dy * dy + dz * dz)


def _full(shape):
    return pl.BlockSpec(shape, lambda *_: (0,) * len(shape))


def _attn_param_list(p):
    return [p['to_qkv'],
            p['pos_w1'], p['pos_b1'].reshape(1, -1),
            p['pos_w2'], p['pos_b2'].reshape(1, -1),
            p['attn_w1'], p['attn_b1'].reshape(1, -1),
            p['attn_w2'], p['attn_b2'].reshape(1, -1)]


def _bd16(w):
    ci, co = w.shape
    eye = jnp.eye(16, dtype=F32)
    return (w[:, None, :, None] * eye[None, :, None, :]).reshape(ci * 16, co * 16)


def _attn_params_m(p):
    s48 = (jnp.eye(3, dtype=F32)[:, None, :, None]
           * jnp.ones((16, 16), F32)[None, :, None, :]).reshape(48, 48)
    r48 = (jnp.eye(3, dtype=F32)[:, None, :]
           * jnp.ones((16,), F32)[None, :, None]).reshape(48, 3)
    return [p['to_qkv'],
            _bd16(p['pos_w1']), jnp.repeat(p['pos_b1'], 16)[None, :],
            _bd16(p['pos_w2']), jnp.repeat(p['pos_b2'], 16)[None, :],
            _bd16(p['attn_w1']), jnp.repeat(p['attn_b1'], 16)[None, :],
            _bd16(p['attn_w2']), jnp.repeat(p['attn_b2'], 16)[None, :],
            s48, r48]


def _perm48():
    r = jnp.arange(48)
    return (r % 16) * 3 + r // 16


def _small_attn_body(nP, nM, *refs):
    x_ref, pos_ref = refs[0], refs[1]
    i = 2 + (1 if nM == 6 else 0)
    P = [r[...] for r in refs[i:i + nP]]
    M = [r[...] for r in refs[i + nP:i + nP + nM]]
    out_ref = refs[-1]
    a = _ptl3_body(x_ref[...], pos_ref[...], P)
    T = a.shape[0]
    o = jnp.swapaxes(a, 1, 2).reshape(T, 48)
    if nM == 4:
        o = _relu(_dot(o, M[0]) + M[1])
        o = _relu(_dot(o, M[2]) + M[3])
    elif nM == 6:
        o = jnp.concatenate([o, refs[2][...]], axis=1)
        o = _relu(_dot(o, M[0]) + M[1])
        o = _relu(_dot(o, M[2]) + M[3])
        o = _dot(o, M[4]) + M[5]
    out_ref[...] = o


def _small_attn_call(x, pos, pA, mlp=None, mlp_out=None, sa2rows=None, T=32):
    G = x.shape[0]
    params = _attn_params_m(pA)
    extra = []
    nM = 0
    if mlp is not None:
        extra = [mlp['w1'][_perm48(), :], mlp['b1'].reshape(1, -1),
                 mlp['w2'], mlp['b2'].reshape(1, -1)]
        nM = 4
    if mlp_out is not None:
        w1 = mlp_out['w1']
        w1 = jnp.concatenate([w1[:48][_perm48(), :], w1[48:]], axis=0)
        extra = [w1, mlp_out['b1'].reshape(1, -1),
                 mlp_out['w2'], mlp_out['b2'].reshape(1, -1),
                 mlp_out['w3'], mlp_out['b3'].reshape(1, -1)]
        nM = 6
    in_specs = [pl.BlockSpec((T, 16, 3), lambda i: (i, 0, 0)),
                pl.BlockSpec((T, 16, 3), lambda i: (i, 0, 0))]
    args = [x, pos]
    if nM == 6:
        in_specs.append(pl.BlockSpec((T, 48), lambda i: (i, 0)))
        args.append(sa2rows)
    in_specs += [_full(p.shape) for p in params + extra]
    args += params + extra
    Fout = 1 if nM == 6 else 48
    return pl.pallas_call(
        functools.partial(_small_attn_body, 11, nM),
        grid=(G // T,),
        in_specs=in_specs,
        out_specs=pl.BlockSpec((T, Fout), lambda i: (i, 0)),
        out_shape=jax.ShapeDtypeStruct((G, Fout), F32),
        interpret=_INTERPRET,
    )(*args)


def _stage_b_body(sa1_ref, pts_ref, ptsT_ref, *rest):
    refs, out_ref = rest[:-1], rest[-1]
    (qkv_w, pw1, pb1, pw2, pb2, aw1, ab1, aw2, ab2,
     mw1, mb1, mw2, mb2) = (r[...] for r in refs)
    qi = pl.program_id(1)
    x = sa1_ref[0]
    pT = ptsT_ref[0]
    p = pts_ref[0]
    kv = _dot(x, qkv_w)
    k, v = kv[:, 48:96], kv[:, 96:144]
    xq = sa1_ref[0, pl.ds(qi * 128, 128), :]
    pq = pts_ref[0, pl.ds(qi * 128, 128), :]
    q = _dot(xq, qkv_w)[:, 0:48]
    dist = _pair_dist(pq, pT)
    feats = jnp.concatenate([k, v, p], axis=1)
    g = _topk16_gather(dist, feats)
    gk, gv, gp = g[:, :, 0:48], g[:, :, 48:96], g[:, :, 96:99]
    grel = pq[:, None, :] - gp
    h = _relu(_dot(grel.reshape(128 * 16, 3), pw1) + pb1)
    emb = (_dot(h, pw2) + pb2).reshape(128, 16, 48)
    qkr = q[:, None, :] - gk
    vg = gv + emb
    h2 = _relu(_dot((qkr + emb).reshape(128 * 16, 48), aw1) + ab1)
    sim = (_dot(h2, aw2) + ab2).reshape(128, 16, 48)
    attn = _softmax_axis(sim, 1)
    o = jnp.sum(attn * vg, axis=1)
    o = _relu(_dot(o, mw1) + mb1)
    o = _relu(_dot(o, mw2) + mb2)
    out_ref[0] = o


def _call_b(sa1, pts, ptsT, pA, pM):
    B = sa1.shape[0]
    params = _attn_param_list(pA) + [pM['w1'], pM['b1'].reshape(1, -1),
                                     pM['w2'], pM['b2'].reshape(1, -1)]
    in_specs = [pl.BlockSpec((1, 512, 48), lambda b, q: (b, 0, 0)),
                pl.BlockSpec((1, 512, 3), lambda b, q: (b, 0, 0)),
                pl.BlockSpec((1, 3, 512), lambda b, q: (b, 0, 0))]
    in_specs += [_full(p.shape) for p in params]
    return pl.pallas_call(
        _stage_b_body,
        grid=(B, 4),
        in_specs=in_specs,
        out_specs=pl.BlockSpec((1, 128, 48), lambda b, q: (b, q, 0)),
        out_shape=jax.ShapeDtypeStruct((B, 512, 48), F32),
        interpret=_INTERPRET,
    )(sa1, pts, ptsT, *params)


def _stage_c1_body(sa2_ref, pts_ref, ptsT_ref, perm_ref,
                   xm_ref, gp_ref, piv_ref):
    s = sa2_ref[0]
    p = pts_ref[0]
    pT = ptsT_ref[0]
    prm = perm_ref[...]
    j512 = jax.lax.broadcasted_iota(jnp.int32, (128, 512), 1)
    ohp = (j512 == prm).astype(F32)
    piv = _dotx(ohp, p)
    dist = _pair_dist(piv, pT)
    feats = jnp.concatenate([s, p], axis=1)
    g = _topk16_gather(dist, feats)
    gs, gp = g[:, :, 0:48], g[:, :, 48:51]
    xm_ref[0] = jnp.max(gs.reshape(128, 16, 3, 16), axis=-1)
    gp_ref[0] = gp
    piv_ref[0] = piv


def _call_c1(sa2, pts, ptsT, permc):
    B = sa2.shape[0]
    in_specs = [pl.BlockSpec((1, 512, 48), lambda b: (b, 0, 0)),
                pl.BlockSpec((1, 512, 3), lambda b: (b, 0, 0)),
                pl.BlockSpec((1, 3, 512), lambda b: (b, 0, 0)),
                _full((128, 1))]
    return pl.pallas_call(
        _stage_c1_body,
        grid=(B,),
        in_specs=in_specs,
        out_specs=[pl.BlockSpec((1, 128, 16, 3), lambda b: (b, 0, 0, 0)),
                   pl.BlockSpec((1, 128, 16, 3), lambda b: (b, 0, 0, 0)),
                   pl.BlockSpec((1, 128, 3), lambda b: (b, 0, 0))],
        out_shape=[jax.ShapeDtypeStruct((B, 128, 16, 3), F32),
                   jax.ShapeDtypeStruct((B, 128, 16, 3), F32),
                   jax.ShapeDtypeStruct((B, 128, 3), F32)],
        interpret=_INTERPRET,
    )(sa2, pts, ptsT, permc)


def _stage_d_body(sa3_ref, piv_ref, *rest):
    refs, out_ref = rest[:-1], rest[-1]
    (qkv_w, pw1, pb1, pw2, pb2, aw1, ab1, aw2, ab2) = (r[...] for r in refs)
    qi = pl.program_id(1)
    x = sa3_ref[0]
    piv = piv_ref[0]
    kv = _dot(x, qkv_w)
    k, v = kv[:, 48:96], kv[:, 96:144]
    xq = sa3_ref[0, pl.ds(qi * 32, 32), :]
    pq = piv_ref[0, pl.ds(qi * 32, 32), :]
    qq = _dot(xq, qkv_w)[:, 0:48]
    rel = pq[:, None, :] - piv[None, :, :]
    h = _relu(_dot(rel.reshape(32 * 128, 3), pw1) + pb1)
    emb = (_dot(h, pw2) + pb2).reshape(32, 128, 48)
    qk = qq[:, None, :] - k[None, :, :]
    vb = v[None, :, :] + emb
    h2 = _relu(_dot((qk + emb).reshape(32 * 128, 48), aw1) + ab1)
    sim = (_dot(h2, aw2) + ab2).reshape(32, 128, 48)
    attn = _softmax_axis(sim, 1)
    out_ref[0] = jnp.sum(attn * vb, axis=1)


def _call_d(sa3, pivot, pA):
    B = sa3.shape[0]
    params = _attn_param_list(pA)
    in_specs = [pl.BlockSpec((1, 128, 48), lambda b, q: (b, 0, 0)),
                pl.BlockSpec((1, 128, 3), lambda b, q: (b, 0, 0))]
    in_specs += [_full(p.shape) for p in params]
    return pl.pallas_call(
        _stage_d_body,
        grid=(B, 4),
        in_specs=in_specs,
        out_specs=pl.BlockSpec((1, 32, 48), lambda b, q: (b, q, 0)),
        out_shape=jax.ShapeDtypeStruct((B, 128, 48), F32),
        interpret=_INTERPRET,
    )(sa3, pivot, *params)


def _stage_e1_body(pts_ref, pivT_ref, piv_ref, sa4_ref, xm_ref, gp_ref):
    qi = pl.program_id(1)
    pvT = pivT_ref[0]
    piv = piv_ref[0]
    s4 = sa4_ref[0]
    pq = pts_ref[0, pl.ds(qi * 128, 128), :]
    pc = jnp.concatenate([pvT, jnp.zeros((3, 384), F32)], axis=1)
    dist = _pair_dist(pq, pc)
    feats = jnp.concatenate([s4, piv], axis=1)
    g = _topk16_gather(dist, feats, clamp=127)
    gs, gp = g[:, :, 0:48], g[:, :, 48:51]
    xm_ref[0] = jnp.max(gs.reshape(128, 16, 3, 16), axis=-1)
    gp_ref[0] = gp


def _call_e1(pts, pivT, pivot, sa4):
    B = pts.shape[0]
    in_specs = [pl.BlockSpec((1, 512, 3), lambda b, q: (b, 0, 0)),
                pl.BlockSpec((1, 3, 128), lambda b, q: (b, 0, 0)),
                pl.BlockSpec((1, 128, 3), lambda b, q: (b, 0, 0)),
                pl.BlockSpec((1, 128, 48), lambda b, q: (b, 0, 0))]
    return pl.pallas_call(
        _stage_e1_body,
        grid=(B, 4),
        in_specs=in_specs,
        out_specs=[pl.BlockSpec((1, 128, 16, 3), lambda b, q: (b, q, 0, 0)),
                   pl.BlockSpec((1, 128, 16, 3), lambda b, q: (b, q, 0, 0))],
        out_shape=[jax.ShapeDtypeStruct((B, 512, 16, 3), F32),
                   jax.ShapeDtypeStruct((B, 512, 16, 3), F32)],
        interpret=_INTERPRET,
    )(pts, pivT, pivot, sa4)


def kernel(original_points, data, perm, params):
    B, N, S, _ = data.shape
    x0 = data.reshape(B * N, S, 3)
    sa1m = _small_attn_call(x0, x0, params['attn1'], mlp=params['mlp'])
    ptsT = jnp.swapaxes(original_points, 1, 2)
    sa2 = _call_b(sa1m.reshape(B, N, 48), original_points, ptsT,
                  params['attn2'], params['mlp'])
    permc = perm[:128].reshape(128, 1)
    cxm, cgp, pivot = _call_c1(sa2, original_points, ptsT, permc)
    sa3 = _small_attn_call(cxm.reshape(B * 128, 16, 3),
                           cgp.reshape(B * 128, 16, 3),
                           params['attn3']).reshape(B, 128, 48)
    pD = dict(params['attn4'])
    pD['to_qkv'] = pD['to_qkv'][_perm48(), :]
    sa4 = _call_d(sa3, pivot, pD)
    pivT = jnp.swapaxes(pivot, 1, 2)
    xm, gp = _call_e1(original_points, pivT, pivot, sa4)
    out = _small_attn_call(xm.reshape(B * N, 16, 3),
                           gp.reshape(B * N, 16, 3),
                           params['attn5'], mlp_out=params['mlp_out'],
                           sa2rows=sa2.reshape(B * N, 48))
    return out.reshape(B, N, 1)


# speedup vs baseline: 2.6604x; 1.2391x over previous
import functools

import jax
import jax.numpy as jnp
from jax.experimental import pallas as pl

F32 = jnp.float32
_INTERPRET = False

_BIG = 1 << 30


def _dot(a, b):
    return jnp.dot(a, b, preferred_element_type=F32)


def _dotx(a, b):
    return jnp.dot(a, b, preferred_element_type=F32,
                   precision=jax.lax.Precision.HIGHEST)


def _doth(a, b_hi, b_lo):
    return (jnp.dot(a, b_hi, preferred_element_type=F32)
            + jnp.dot(a, b_lo, preferred_element_type=F32))


def _relu(x):
    return jnp.maximum(x, 0.0)


def _softmax_axis(x, axis):
    m = jnp.max(x, axis=axis, keepdims=True)
    e = jnp.exp(x - m)
    return e / jnp.sum(e, axis=axis, keepdims=True)


def _ptl3_body(x, pos, xJ, posJ, P):
    T = x.shape[0]
    qkv = _dot(x.reshape(T * 16, 3), P[0])
    q = qkv[:, 0:3]
    qkvJ = _dot(xJ, P[11])
    kJ = qkvJ[:, 48:96].reshape(T, 1, 48)
    vJ = qkvJ[:, 96:144].reshape(T, 1, 48)
    posI = _dotx(pos.reshape(T * 16, 3), P[12]).reshape(T, 16, 48)
    qI = _dotx(q, P[12]).reshape(T, 16, 48)
    rel = (posI - posJ.reshape(T, 1, 48)).reshape(T * 16, 48)
    h = _relu(_dot(rel, P[1]) + P[2])
    emb = _dot(h, P[3]) + P[4]
    qk = (qI - kJ).reshape(T * 16, 48)
    vb = vJ + emb.reshape(T, 16, 48)
    h2 = _relu(_dot(qk + emb, P[5]) + P[6])
    sim = (_dot(h2, P[7]) + P[8]).reshape(T, 16, 48)
    m = jnp.max(sim, axis=2, keepdims=True)
    e = jnp.exp(sim - m)
    esum = _dotx(e.reshape(T * 16, 48), P[9]).reshape(T, 16, 48)
    attn = e / esum
    return _dotx((attn * vb).reshape(T * 16, 48), P[10]).reshape(T, 16, 3)


def _topk16_gather(dist, feats, clamp=None):
    R, N = dist.shape
    C = feats.shape[0]
    neg = -dist
    jcol = jax.lax.broadcasted_iota(jnp.int32, (R, N), 1)
    jg = jcol if clamp is None else jax.lax.broadcasted_iota(jnp.int32, (R, C), 1)
    f_hi = feats.astype(jnp.bfloat16).astype(F32)
    f_lo = feats - f_hi
    outs = []
    for _ in range(16):
        m = jnp.max(neg, axis=1, keepdims=True)
        cand = jnp.where(neg == m, jcol, _BIG)
        jmin = jnp.min(cand, axis=1, keepdims=True)
        ohb = jcol == jmin
        neg = jnp.where(ohb, -jnp.inf, neg)
        if clamp is not None:
            ohb = jg == jnp.minimum(jmin, clamp)
        oh = ohb.astype(F32)
        outs.append(_doth(oh, f_hi, f_lo))
    return jnp.stack(outs, axis=1)


def _pair_dist(p, pT):
    dx = p[:, 0:1] - pT[0:1, :]
    dy = p[:, 1:2] - pT[1:2, :]
    dz = p[:, 2:3] - pT[2:3, :]
    return jnp.sqrt(dx * dx + dy * dy + dz * dz)


def _full(shape):
    return pl.BlockSpec(shape, lambda *_: (0,) * len(shape))


def _attn_param_list(p):
    return [p['to_qkv'],
            p['pos_w1'], p['pos_b1'].reshape(1, -1),
            p['pos_w2'], p['pos_b2'].reshape(1, -1),
            p['attn_w1'], p['attn_b1'].reshape(1, -1),
            p['attn_w2'], p['attn_b2'].reshape(1, -1)]


def _bd16(w):
    ci, co = w.shape
    eye = jnp.eye(16, dtype=F32)
    return (w[:, None, :, None] * eye[None, :, None, :]).reshape(ci * 16, co * 16)


def _attn_params_m(p):
    s48 = (jnp.eye(3, dtype=F32)[:, None, :, None]
           * jnp.ones((16, 16), F32)[None, :, None, :]).reshape(48, 48)
    r48 = (jnp.eye(3, dtype=F32)[:, None, :]
           * jnp.ones((16,), F32)[None, :, None]).reshape(48, 3)
    e348 = (jnp.eye(3, dtype=F32)[:, :, None]
            * jnp.ones((16,), F32)[None, None, :]).reshape(3, 48)
    return [p['to_qkv'],
            _bd16(p['pos_w1']), jnp.repeat(p['pos_b1'], 16)[None, :],
            _bd16(p['pos_w2']), jnp.repeat(p['pos_b2'], 16)[None, :],
            _bd16(p['attn_w1']), jnp.repeat(p['attn_b1'], 16)[None, :],
            _bd16(p['attn_w2']), jnp.repeat(p['attn_b2'], 16)[None, :],
            s48, r48, _bd16(p['to_qkv']), e348]


def _perm48():
    r = jnp.arange(48)
    return (r % 16) * 3 + r // 16


def _small_attn_body(nP, nM, *refs):
    x_ref, pos_ref, xJ_ref, posJ_ref = refs[0], refs[1], refs[2], refs[3]
    i = 4 + (1 if nM == 6 else 0)
    P = [r[...] for r in refs[i:i + nP]]
    M = [r[...] for r in refs[i + nP:i + nP + nM]]
    out_ref = refs[-1]
    a = _ptl3_body(x_ref[...], pos_ref[...], xJ_ref[...], posJ_ref[...], P)
    T = a.shape[0]
    o = jnp.swapaxes(a, 1, 2).reshape(T, 48)
    if nM == 4:
        o = _relu(_dot(o, M[0]) + M[1])
        o = _relu(_dot(o, M[2]) + M[3])
    elif nM == 6:
        o = jnp.concatenate([o, refs[4][...]], axis=1)
        o = _relu(_dot(o, M[0]) + M[1])
        o = _relu(_dot(o, M[2]) + M[3])
        o = _dot(o, M[4]) + M[5]
    out_ref[...] = o


def _small_attn_call(x, pos, pA, mlp=None, mlp_out=None, sa2rows=None, T=64):
    G = x.shape[0]
    params = _attn_params_m(pA)
    xJ = jnp.swapaxes(x, 1, 2).reshape(G, 48)
    posJ = xJ if pos is x else jnp.swapaxes(pos, 1, 2).reshape(G, 48)
    extra = []
    nM = 0
    if mlp is not None:
        extra = [mlp['w1'][_perm48(), :], mlp['b1'].reshape(1, -1),
                 mlp['w2'], mlp['b2'].reshape(1, -1)]
        nM = 4
    if mlp_out is not None:
        w1 = mlp_out['w1']
        w1 = jnp.concatenate([w1[:48][_perm48(), :], w1[48:]], axis=0)
        extra = [w1, mlp_out['b1'].reshape(1, -1),
                 mlp_out['w2'], mlp_out['b2'].reshape(1, -1),
                 mlp_out['w3'], mlp_out['b3'].reshape(1, -1)]
        nM = 6
    in_specs = [pl.BlockSpec((T, 16, 3), lambda i: (i, 0, 0)),
                pl.BlockSpec((T, 16, 3), lambda i: (i, 0, 0)),
                pl.BlockSpec((T, 48), lambda i: (i, 0)),
                pl.BlockSpec((T, 48), lambda i: (i, 0))]
    args = [x, pos, xJ, posJ]
    if nM == 6:
        in_specs.append(pl.BlockSpec((T, 48), lambda i: (i, 0)))
        args.append(sa2rows)
    in_specs += [_full(p.shape) for p in params + extra]
    args += params + extra
    Fout = 1 if nM == 6 else 48
    return pl.pallas_call(
        functools.partial(_small_attn_body, 13, nM),
        grid=(G // T,),
        in_specs=in_specs,
        out_specs=pl.BlockSpec((T, Fout), lambda i: (i, 0)),
        out_shape=jax.ShapeDtypeStruct((G, Fout), F32),
        interpret=_INTERPRET,
    )(*args)


def _stage_b_body(sa1_ref, pts_ref, ptsT_ref, *rest):
    refs, out_ref = rest[:-1], rest[-1]
    (qkv_w, pw1, pb1, pw2, pb2, aw1, ab1, aw2, ab2,
     mw1, mb1, mw2, mb2) = (r[...] for r in refs)
    qi = pl.program_id(1)
    x = sa1_ref[0]
    pT = ptsT_ref[0]
    p = pts_ref[0]
    kv = _dot(x, qkv_w)
    k, v = kv[:, 48:96], kv[:, 96:144]
    xq = sa1_ref[0, pl.ds(qi * 128, 128), :]
    pq = pts_ref[0, pl.ds(qi * 128, 128), :]
    q = _dot(xq, qkv_w)[:, 0:48]
    dist = _pair_dist(pq, pT)
    feats = jnp.concatenate([k, v, p], axis=1)
    g = _topk16_gather(dist, feats)
    gk, gv, gp = g[:, :, 0:48], g[:, :, 48:96], g[:, :, 96:99]
    grel = pq[:, None, :] - gp
    h = _relu(_dot(grel.reshape(128 * 16, 3), pw1) + pb1)
    emb = (_dot(h, pw2) + pb2).reshape(128, 16, 48)
    qkr = q[:, None, :] - gk
    vg = gv + emb
    h2 = _relu(_dot((qkr + emb).reshape(128 * 16, 48), aw1) + ab1)
    sim = (_dot(h2, aw2) + ab2).reshape(128, 16, 48)
    attn = _softmax_axis(sim, 1)
    o = jnp.sum(attn * vg, axis=1)
    o = _relu(_dot(o, mw1) + mb1)
    o = _relu(_dot(o, mw2) + mb2)
    out_ref[0] = o


def _call_b(sa1, pts, ptsT, pA, pM):
    B = sa1.shape[0]
    params = _attn_param_list(pA) + [pM['w1'], pM['b1'].reshape(1, -1),
                                     pM['w2'], pM['b2'].reshape(1, -1)]
    in_specs = [pl.BlockSpec((1, 512, 48), lambda b, q: (b, 0, 0)),
                pl.BlockSpec((1, 512, 3), lambda b, q: (b, 0, 0)),
                pl.BlockSpec((1, 3, 512), lambda b, q: (b, 0, 0))]
    in_specs += [_full(p.shape) for p in params]
    return pl.pallas_call(
        _stage_b_body,
        grid=(B, 4),
        in_specs=in_specs,
        out_specs=pl.BlockSpec((1, 128, 48), lambda b, q: (b, q, 0)),
        out_shape=jax.ShapeDtypeStruct((B, 512, 48), F32),
        interpret=_INTERPRET,
    )(sa1, pts, ptsT, *params)


def _stage_c1_body(sa2_ref, pts_ref, ptsT_ref, perm_ref,
                   xm_ref, gp_ref, piv_ref):
    s = sa2_ref[0]
    p = pts_ref[0]
    pT = ptsT_ref[0]
    prm = perm_ref[...]
    j512 = jax.lax.broadcasted_iota(jnp.int32, (128, 512), 1)
    ohp = (j512 == prm).astype(F32)
    piv = _dotx(ohp, p)
    dist = _pair_dist(piv, pT)
    sp = jnp.max(s.reshape(512, 3, 16), axis=-1)
    feats = jnp.concatenate([sp, p], axis=1)
    g = _topk16_gather(dist, feats)
    xm_ref[0] = g[:, :, 0:3]
    gp_ref[0] = g[:, :, 3:6]
    piv_ref[0] = piv


def _call_c1(sa2, pts, ptsT, permc):
    B = sa2.shape[0]
    in_specs = [pl.BlockSpec((1, 512, 48), lambda b: (b, 0, 0)),
                pl.BlockSpec((1, 512, 3), lambda b: (b, 0, 0)),
                pl.BlockSpec((1, 3, 512), lambda b: (b, 0, 0)),
                _full((128, 1))]
    return pl.pallas_call(
        _stage_c1_body,
        grid=(B,),
        in_specs=in_specs,
        out_specs=[pl.BlockSpec((1, 128, 16, 3), lambda b: (b, 0, 0, 0)),
                   pl.BlockSpec((1, 128, 16, 3), lambda b: (b, 0, 0, 0)),
                   pl.BlockSpec((1, 128, 3), lambda b: (b, 0, 0))],
        out_shape=[jax.ShapeDtypeStruct((B, 128, 16, 3), F32),
                   jax.ShapeDtypeStruct((B, 128, 16, 3), F32),
                   jax.ShapeDtypeStruct((B, 128, 3), F32)],
        interpret=_INTERPRET,
    )(sa2, pts, ptsT, permc)


def _stage_d_body(sa3_ref, piv_ref, *rest):
    refs, out_ref = rest[:-1], rest[-1]
    (qkv_w, pw1, pb1, pw2, pb2, aw1, ab1, aw2, ab2) = (r[...] for r in refs)
    qi = pl.program_id(1)
    x = sa3_ref[0]
    piv = piv_ref[0]
    kv = _dot(x, qkv_w)
    k, v = kv[:, 48:96], kv[:, 96:144]
    xq = sa3_ref[0, pl.ds(qi * 32, 32), :]
    pq = piv_ref[0, pl.ds(qi * 32, 32), :]
    qq = _dot(xq, qkv_w)[:, 0:48]
    rel = pq[:, None, :] - piv[None, :, :]
    h = _relu(_dot(rel.reshape(32 * 128, 3), pw1) + pb1)
    emb = (_dot(h, pw2) + pb2).reshape(32, 128, 48)
    qk = qq[:, None, :] - k[None, :, :]
    vb = v[None, :, :] + emb
    h2 = _relu(_dot((qk + emb).reshape(32 * 128, 48), aw1) + ab1)
    sim = (_dot(h2, aw2) + ab2).reshape(32, 128, 48)
    attn = _softmax_axis(sim, 1)
    out_ref[0] = jnp.sum(attn * vb, axis=1)


def _call_d(sa3, pivot, pA):
    B = sa3.shape[0]
    params = _attn_param_list(pA)
    in_specs = [pl.BlockSpec((1, 128, 48), lambda b, q: (b, 0, 0)),
                pl.BlockSpec((1, 128, 3), lambda b, q: (b, 0, 0))]
    in_specs += [_full(p.shape) for p in params]
    return pl.pallas_call(
        _stage_d_body,
        grid=(B, 4),
        in_specs=in_specs,
        out_specs=pl.BlockSpec((1, 32, 48), lambda b, q: (b, q, 0)),
        out_shape=jax.ShapeDtypeStruct((B, 128, 48), F32),
        interpret=_INTERPRET,
    )(sa3, pivot, *params)


def _stage_e1_body(pts_ref, pivT_ref, piv_ref, sa4_ref, xm_ref, gp_ref):
    qi = pl.program_id(1)
    pvT = pivT_ref[0]
    piv = piv_ref[0]
    s4 = sa4_ref[0]
    pq = pts_ref[0, pl.ds(qi * 128, 128), :]
    pc = jnp.concatenate([pvT, jnp.zeros((3, 384), F32)], axis=1)
    dist = _pair_dist(pq, pc)
    s4p = jnp.max(s4.reshape(128, 3, 16), axis=-1)
    feats = jnp.concatenate([s4p, piv], axis=1)
    g = _topk16_gather(dist, feats, clamp=127)
    xm_ref[0] = g[:, :, 0:3]
    gp_ref[0] = g[:, :, 3:6]


def _call_e1(pts, pivT, pivot, sa4):
    B = pts.shape[0]
    in_specs = [pl.BlockSpec((1, 512, 3), lambda b, q: (b, 0, 0)),
                pl.BlockSpec((1, 3, 128), lambda b, q: (b, 0, 0)),
                pl.BlockSpec((1, 128, 3), lambda b, q: (b, 0, 0)),
                pl.BlockSpec((1, 128, 48), lambda b, q: (b, 0, 0))]
    return pl.pallas_call(
        _stage_e1_body,
        grid=(B, 4),
        in_specs=in_specs,
        out_specs=[pl.BlockSpec((1, 128, 16, 3), lambda b, q: (b, q, 0, 0)),
                   pl.BlockSpec((1, 128, 16, 3), lambda b, q: (b, q, 0, 0))],
        out_shape=[jax.ShapeDtypeStruct((B, 512, 16, 3), F32),
                   jax.ShapeDtypeStruct((B, 512, 16, 3), F32)],
        interpret=_INTERPRET,
    )(pts, pivT, pivot, sa4)


def kernel(original_points, data, perm, params):
    B, N, S, _ = data.shape
    x0 = data.reshape(B * N, S, 3)
    sa1m = _small_attn_call(x0, x0, params['attn1'], mlp=params['mlp'])
    ptsT = jnp.swapaxes(original_points, 1, 2)
    sa2 = _call_b(sa1m.reshape(B, N, 48), original_points, ptsT,
                  params['attn2'], params['mlp'])
    permc = perm[:128].reshape(128, 1)
    cxm, cgp, pivot = _call_c1(sa2, original_points, ptsT, permc)
    sa3 = _small_attn_call(cxm.reshape(B * 128, 16, 3),
                           cgp.reshape(B * 128, 16, 3),
                           params['attn3']).reshape(B, 128, 48)
    pD = dict(params['attn4'])
    pD['to_qkv'] = pD['to_qkv'][_perm48(), :]
    sa4 = _call_d(sa3, pivot, pD)
    pivT = jnp.swapaxes(pivot, 1, 2)
    xm, gp = _call_e1(original_points, pivT, pivot, sa4)
    out = _small_attn_call(xm.reshape(B * N, 16, 3),
                           gp.reshape(B * N, 16, 3),
                           params['attn5'], mlp_out=params['mlp_out'],
                           sa2rows=sa2.reshape(B * N, 48))
    return out.reshape(B, N, 1)
